# Initial kernel scaffold; baseline (speedup 1.0000x reference)
import jax
import jax.numpy as jnp
from jax.experimental import pallas as pl


def kernel(seed_map, offset_map, labels, sigma_map):
    raise NotImplementedError("write your pallas kernel here")


# speedup vs baseline: 5.9783x; 5.9783x over previous
import jax
import jax.numpy as jnp
from jax.experimental import pallas as pl

_H, _W = 384, 512
_B, _C, _S = 2, 2, 2
_NUM_IDS = 9
_I = _NUM_IDS - 1
_SCALE = 64.0
_RESCALE = 1.0 / 64.0
_EMB = (64.0, 64.0)


def _plane_kernel(seed_ref, off_ref, lab_ref, sig_ref, dists_ref, sums_ref):
    b = pl.program_id(0)
    c = pl.program_id(1)
    lab = lab_ref[0, 0]
    sig = _SCALE / (1.0 + jnp.exp(sig_ref[0, 0] * (-1.0 / (2.0 * _SCALE))))
    row = jax.lax.broadcasted_iota(jnp.int32, (_H, _W), 0).astype(jnp.float32)
    col = jax.lax.broadcasted_iota(jnp.int32, (_H, _W), 1).astype(jnp.float32)
    e0 = (row * (_EMB[0] / _H) + off_ref[0, 0]) * _RESCALE
    e1 = (col * (_EMB[1] / _W) + off_ref[0, 1]) * _RESCALE
    seed = seed_ref[0, 0]

    inst_sum = jnp.zeros((), jnp.float32)
    smooth_sum = jnp.zeros((), jnp.float32)
    seed_mask = jnp.zeros((_H, _W), jnp.float32)
    for i in range(_I):
        m = lab == (i + 1)
        mf = m.astype(jnp.float32)
        cnt = jnp.sum(mf)
        sm = jnp.sum(jnp.where(m, sig, 0.0)) / cnt
        m0 = jnp.sum(jnp.where(m, e0, 0.0)) / cnt
        m1 = jnp.sum(jnp.where(m, e1, 0.0)) / cnt
        inv2s2 = 1.0 / (2.0 * sm * sm)
        d0 = e0 - m0
        d1 = e1 - m1
        d = (d0 * d0 + d1 * d1) * inv2s2
        phi = jnp.exp(-d)
        dists_ref[0, i] = phi
        logp = jnp.maximum(jnp.log(phi), -100.0)
        log1mp = jnp.maximum(jnp.log(1.0 - phi), -100.0)
        inst_sum = inst_sum - jnp.sum(mf * logp + (1.0 - mf) * log1mp)
        ds = sig - sm
        smooth_sum = smooth_sum + jnp.sum(jnp.where(m, ds * ds, 0.0)) / cnt
        seed_mask = seed_mask + jnp.where(m, phi, 0.0)

    dseed = seed - seed_mask
    seed_sum = jnp.sum(dseed * dseed) * (1.0 / (_H * _W))

    cur = jnp.stack([
        jnp.full((8, 128), inst_sum, jnp.float32),
        jnp.full((8, 128), smooth_sum, jnp.float32),
        jnp.full((8, 128), seed_sum, jnp.float32),
    ])
    first = jnp.logical_and(b == 0, c == 0)

    @pl.when(first)
    def _init():
        sums_ref[...] = cur

    @pl.when(jnp.logical_not(first))
    def _acc():
        sums_ref[...] = sums_ref[...] + cur


def kernel(seed_map, offset_map, labels, sigma_map):
    labels = labels.astype(jnp.int32)
    dists, sums = pl.pallas_call(
        _plane_kernel,
        grid=(_B, _C),
        in_specs=[
            pl.BlockSpec((1, 1, _H, _W), lambda b, c: (b, c, 0, 0)),
            pl.BlockSpec((1, _S, _H, _W), lambda b, c: (b, 0, 0, 0)),
            pl.BlockSpec((1, 1, _H, _W), lambda b, c: (b, c, 0, 0)),
            pl.BlockSpec((1, 1, _H, _W), lambda b, c: (b, 0, 0, 0)),
        ],
        out_specs=[
            pl.BlockSpec((1, _I, _H, _W), lambda b, c: (b, 0, 0, 0)),
            pl.BlockSpec((3, 8, 128), lambda b, c: (0, 0, 0)),
        ],
        out_shape=[
            jax.ShapeDtypeStruct((_B, _I, _H, _W), jnp.float32),
            jax.ShapeDtypeStruct((3, 8, 128), jnp.float32),
        ],
    )(seed_map, offset_map, labels, sigma_map)

    s = sums[:, 0, 0]
    n = float(_B * _C * _I)
    il = s[0] / n
    sl = s[1] / n
    sel = s[2]
    loss = (il + sl + sel) * (1.0 / (_B * _C))
    stats = jnp.stack([il, sl, sel])
    return loss, dists, stats


# speedup vs baseline: 6.5259x; 1.0916x over previous
import jax
import jax.numpy as jnp
from jax.experimental import pallas as pl

_H, _W = 384, 512
_B, _C, _S = 2, 2, 2
_NUM_IDS = 9
_I = _NUM_IDS - 1
_SCALE = 64.0
_RESCALE = 1.0 / 64.0
_EMB = (64.0, 64.0)


def _plane_kernel(seed_ref, off_ref, lab_ref, sig_ref, dists_ref, sums_ref):
    b = pl.program_id(0)
    c = pl.program_id(1)
    lab = lab_ref[0, 0]
    sig = _SCALE / (1.0 + jnp.exp(sig_ref[0, 0] * (-1.0 / (2.0 * _SCALE))))
    row = jax.lax.broadcasted_iota(jnp.int32, (_H, _W), 0).astype(jnp.float32)
    col = jax.lax.broadcasted_iota(jnp.int32, (_H, _W), 1).astype(jnp.float32)
    e0 = (row * (_EMB[0] / _H) + off_ref[0, 0]) * _RESCALE
    e1 = (col * (_EMB[1] / _W) + off_ref[0, 1]) * _RESCALE
    seed = seed_ref[0, 0]

    sig2 = sig * sig
    inst_sum = jnp.zeros((), jnp.float32)
    smooth_sum = jnp.zeros((), jnp.float32)
    seed_mask = jnp.zeros((_H, _W), jnp.float32)
    for i in range(_I):
        m = lab == (i + 1)
        cnt = jnp.sum(jnp.where(m, 1.0, 0.0))
        sm = jnp.sum(jnp.where(m, sig, 0.0)) / cnt
        ssq = jnp.sum(jnp.where(m, sig2, 0.0))
        m0 = jnp.sum(jnp.where(m, e0, 0.0)) / cnt
        m1 = jnp.sum(jnp.where(m, e1, 0.0)) / cnt
        inv2s2 = 1.0 / (2.0 * sm * sm)
        d0 = e0 - m0
        d1 = e1 - m1
        d = (d0 * d0 + d1 * d1) * inv2s2
        phi = jnp.exp(-d)

        @pl.when(c == _C - 1)
        def _store():
            dists_ref[0, i] = phi

        logp = jnp.maximum(-d, -100.0)
        log1mp = jnp.maximum(jnp.log(1.0 - phi), -100.0)
        inst_sum = inst_sum - (jnp.sum(log1mp)
                               + jnp.sum(jnp.where(m, logp - log1mp, 0.0)))
        smooth_sum = smooth_sum + (ssq / cnt - sm * sm)
        seed_mask = seed_mask + jnp.where(m, phi, 0.0)

    dseed = seed - seed_mask
    seed_sum = jnp.sum(dseed * dseed) * (1.0 / (_H * _W))

    cur = jnp.stack([
        jnp.full((8, 128), inst_sum, jnp.float32),
        jnp.full((8, 128), smooth_sum, jnp.float32),
        jnp.full((8, 128), seed_sum, jnp.float32),
    ])
    first = jnp.logical_and(b == 0, c == 0)

    @pl.when(first)
    def _init():
        sums_ref[...] = cur

    @pl.when(jnp.logical_not(first))
    def _acc():
        sums_ref[...] = sums_ref[...] + cur


def kernel(seed_map, offset_map, labels, sigma_map):
    labels = labels.astype(jnp.int32)
    dists, sums = pl.pallas_call(
        _plane_kernel,
        grid=(_B, _C),
        in_specs=[
            pl.BlockSpec((1, 1, _H, _W), lambda b, c: (b, c, 0, 0)),
            pl.BlockSpec((1, _S, _H, _W), lambda b, c: (b, 0, 0, 0)),
            pl.BlockSpec((1, 1, _H, _W), lambda b, c: (b, c, 0, 0)),
            pl.BlockSpec((1, 1, _H, _W), lambda b, c: (b, 0, 0, 0)),
        ],
        out_specs=[
            pl.BlockSpec((1, _I, _H, _W), lambda b, c: (b, 0, 0, 0)),
            pl.BlockSpec((3, 8, 128), lambda b, c: (0, 0, 0)),
        ],
        out_shape=[
            jax.ShapeDtypeStruct((_B, _I, _H, _W), jnp.float32),
            jax.ShapeDtypeStruct((3, 8, 128), jnp.float32),
        ],
    )(seed_map, offset_map, labels, sigma_map)

    s = sums[:, 0, 0]
    n = float(_B * _C * _I)
    il = s[0] / n
    sl = s[1] / n
    sel = s[2]
    loss = (il + sl + sel) * (1.0 / (_B * _C))
    stats = jnp.stack([il, sl, sel])
    return loss, dists, stats
